# baseline (speedup 1.0000x reference)
import jax
import jax.numpy as jnp
from jax.experimental import pallas as pl
from jax.experimental.pallas import tpu as pltpu

TOKENS = 16384
HIDDEN = 4096
CTX = HIDDEN // 4
EXPERTS = 64
BM = 1024
WCH = 512


def _router_body(t_ref, x_ref, w1_ref, b1_ref, w2_ref, out_ref,
                 w1b_ref, w2b_ref, stage_ref, sem):
    @pl.when(pl.program_id(0) == 0)
    def _cast_weights():
        for c in range(HIDDEN // WCH):
            cp = pltpu.make_async_copy(
                w1_ref.at[pl.ds(c * WCH, WCH), :], stage_ref, sem)
            cp.start()
            cp.wait()
            w1b_ref[pl.ds(c * WCH, WCH), :] = stage_ref[...].astype(jnp.bfloat16)
        w2b_ref[...] = (w2_ref[...] * (0.5 / t_ref[0])).astype(jnp.bfloat16)

    xb = x_ref[...].astype(jnp.bfloat16)
    h = jnp.dot(xb, w1b_ref[...], preferred_element_type=jnp.float32)
    h = h + b1_ref[...]
    g2 = h * (1.0 + jax.lax.erf(h * 0.7071067811865476))
    out_ref[...] = jnp.dot(g2.astype(jnp.bfloat16), w2b_ref[...],
                           preferred_element_type=jnp.float32)


def kernel(x, W1, b1, W2, temperature):
    b1r = b1.reshape(1, CTX)
    grid = (TOKENS // BM,)
    return pl.pallas_call(
        _router_body,
        grid=grid,
        in_specs=[
            pl.BlockSpec(memory_space=pltpu.SMEM),
            pl.BlockSpec((BM, HIDDEN), lambda i: (i, 0)),
            pl.BlockSpec(memory_space=pl.ANY),
            pl.BlockSpec((1, CTX), lambda i: (0, 0)),
            pl.BlockSpec((CTX, EXPERTS), lambda i: (0, 0)),
        ],
        out_specs=pl.BlockSpec((BM, EXPERTS), lambda i: (i, 0)),
        out_shape=jax.ShapeDtypeStruct((TOKENS, EXPERTS), jnp.float32),
        scratch_shapes=[
            pltpu.VMEM((HIDDEN, CTX), jnp.bfloat16),
            pltpu.VMEM((CTX, EXPERTS), jnp.bfloat16),
            pltpu.VMEM((WCH, CTX), jnp.float32),
            pltpu.SemaphoreType.DMA,
        ],
        compiler_params=pltpu.CompilerParams(
            dimension_semantics=("arbitrary",),
            vmem_limit_bytes=64 * 1024 * 1024,
        ),
    )(temperature, x, W1, b1r, W2)


# speedup vs baseline: 1.0257x; 1.0257x over previous
import jax
import jax.numpy as jnp
from jax.experimental import pallas as pl
from jax.experimental.pallas import tpu as pltpu

TOKENS = 16384
HIDDEN = 4096
CTX = HIDDEN // 4
EXPERTS = 64
BM = 512
NBLK = TOKENS // BM


def _router_body(t_ref, x_ref, w1_ref, b1_ref, w2_ref, out_ref,
                 w1b_ref, w2b_ref, h_ref):
    @pl.when(pl.program_id(0) == 0)
    def _cast_weights():
        w1b_ref[...] = w1_ref[...].astype(jnp.bfloat16)
        w2b_ref[...] = (w2_ref[...] * (0.5 / t_ref[0])).astype(jnp.bfloat16)

    h = h_ref[...] + b1_ref[...]
    g2 = h * (1.0 + jax.lax.erf(h * 0.7071067811865476))
    out_ref[...] = jnp.dot(g2.astype(jnp.bfloat16), w2b_ref[...],
                           preferred_element_type=jnp.float32)

    xb = x_ref[...].astype(jnp.bfloat16)
    h_ref[...] = jnp.dot(xb, w1b_ref[...], preferred_element_type=jnp.float32)


def kernel(x, W1, b1, W2, temperature):
    b1r = b1.reshape(1, CTX)
    grid = (NBLK + 1,)
    return pl.pallas_call(
        _router_body,
        grid=grid,
        in_specs=[
            pl.BlockSpec(memory_space=pltpu.SMEM),
            pl.BlockSpec((BM, HIDDEN),
                         lambda i: (jnp.minimum(i, NBLK - 1), 0)),
            pl.BlockSpec((HIDDEN, CTX), lambda i: (0, 0)),
            pl.BlockSpec((1, CTX), lambda i: (0, 0)),
            pl.BlockSpec((CTX, EXPERTS), lambda i: (0, 0)),
        ],
        out_specs=pl.BlockSpec((BM, EXPERTS),
                               lambda i: (jnp.maximum(i - 1, 0), 0)),
        out_shape=jax.ShapeDtypeStruct((TOKENS, EXPERTS), jnp.float32),
        scratch_shapes=[
            pltpu.VMEM((HIDDEN, CTX), jnp.bfloat16),
            pltpu.VMEM((CTX, EXPERTS), jnp.bfloat16),
            pltpu.VMEM((BM, CTX), jnp.float32),
        ],
        compiler_params=pltpu.CompilerParams(
            dimension_semantics=("arbitrary",),
            vmem_limit_bytes=64 * 1024 * 1024,
        ),
    )(temperature, x, W1, b1r, W2)


# speedup vs baseline: 1.0500x; 1.0237x over previous
import jax
import jax.numpy as jnp
from jax.experimental import pallas as pl
from jax.experimental.pallas import tpu as pltpu

TOKENS = 16384
HIDDEN = 4096
CTX = HIDDEN // 4
EXPERTS = 64
BM = 512


def _router_body(t_ref, x_ref, w1_ref, b1_ref, w2_ref, out_ref,
                 w1b_ref, w2b_ref):
    @pl.when(pl.program_id(0) == 0)
    def _cast_weights():
        w1b_ref[...] = w1_ref[...].astype(jnp.bfloat16)
        w2b_ref[...] = (w2_ref[...] * (0.5 / t_ref[0])).astype(jnp.bfloat16)

    xb = x_ref[...].astype(jnp.bfloat16)
    h = jnp.dot(xb, w1b_ref[...], preferred_element_type=jnp.float32)
    h = h + b1_ref[...]
    g2 = h * (1.0 + jax.lax.erf(h * 0.7071067811865476))
    out_ref[...] = jnp.dot(g2.astype(jnp.bfloat16), w2b_ref[...],
                           preferred_element_type=jnp.float32)


def kernel(x, W1, b1, W2, temperature):
    b1r = b1.reshape(1, CTX)
    grid = (TOKENS // BM,)
    return pl.pallas_call(
        _router_body,
        grid=grid,
        in_specs=[
            pl.BlockSpec(memory_space=pltpu.SMEM),
            pl.BlockSpec((BM, HIDDEN), lambda i: (i, 0)),
            pl.BlockSpec((HIDDEN, CTX), lambda i: (0, 0)),
            pl.BlockSpec((1, CTX), lambda i: (0, 0)),
            pl.BlockSpec((CTX, EXPERTS), lambda i: (0, 0)),
        ],
        out_specs=pl.BlockSpec((BM, EXPERTS), lambda i: (i, 0)),
        out_shape=jax.ShapeDtypeStruct((TOKENS, EXPERTS), jnp.float32),
        scratch_shapes=[
            pltpu.VMEM((HIDDEN, CTX), jnp.bfloat16),
            pltpu.VMEM((CTX, EXPERTS), jnp.bfloat16),
        ],
        compiler_params=pltpu.CompilerParams(
            dimension_semantics=("arbitrary",),
            vmem_limit_bytes=64 * 1024 * 1024,
        ),
    )(temperature, x, W1, b1r, W2)
